# baseline (speedup 1.0000x reference)
import jax
import jax.numpy as jnp
from jax.experimental import pallas as pl
from jax.experimental.pallas import tpu as pltpu

T = 8192
D = 2048
E = 64
K = 8
BT = 2048
DH = D // 2


def _router_block(xa_ref, xb_ref, w_ref, b_ref, router_ref, indices_ref):
    w = w_ref[...]
    dims = (((1,), (1,)), ((), ()))
    scores = (
        jax.lax.dot_general(w[:, :DH], xa_ref[...], dims,
                            preferred_element_type=jnp.float32)
        + jax.lax.dot_general(w[:, DH:], xb_ref[...], dims,
                              preferred_element_type=jnp.float32)
        + b_ref[...]
    )

    eidx = jax.lax.broadcasted_iota(jnp.int32, scores.shape, 0)
    active = jnp.ones(scores.shape, dtype=jnp.bool_)
    neg_inf = jnp.float32(-jnp.inf)
    rowmax = None
    for it in range(K):
        masked = jnp.where(active, scores, neg_inf)
        m = jnp.max(masked, axis=0, keepdims=True)
        if it == 0:
            rowmax = m
        is_m = active & (scores == m)
        cand = jnp.where(is_m, eidx, E)
        j = jnp.min(cand, axis=0, keepdims=True)
        active = active & (eidx != j)
    keep = jnp.logical_not(active)

    expv = jnp.where(keep, jnp.exp(scores - rowmax), 0.0)
    router = expv / jnp.sum(expv, axis=0, keepdims=True)
    router_ref[...] = router.T
    indices_ref[...] = keep.astype(jnp.float32).T


def kernel(inputs, W, b):
    b2 = b.reshape(E, 1)
    grid = (T // BT,)
    router, indices = pl.pallas_call(
        _router_block,
        grid=grid,
        in_specs=[
            pl.BlockSpec((BT, DH), lambda i: (i, 0)),
            pl.BlockSpec((BT, DH), lambda i: (i, 1)),
            pl.BlockSpec((E, D), lambda i: (0, 0)),
            pl.BlockSpec((E, 1), lambda i: (0, 0)),
        ],
        out_specs=[
            pl.BlockSpec((BT, E), lambda i: (i, 0)),
            pl.BlockSpec((BT, E), lambda i: (i, 0)),
        ],
        out_shape=[
            jax.ShapeDtypeStruct((T, E), jnp.float32),
            jax.ShapeDtypeStruct((T, E), jnp.float32),
        ],
        compiler_params=pltpu.CompilerParams(
            dimension_semantics=("parallel",),
        ),
    )(inputs, inputs, W, b2)
    return (router, indices)


# speedup vs baseline: 1.2878x; 1.2878x over previous
import jax
import jax.numpy as jnp
from jax.experimental import pallas as pl
from jax.experimental.pallas import tpu as pltpu

T = 8192
D = 2048
E = 64
K = 8
BT = 2048
DH = D // 2


def _router_block(xa_ref, xb_ref, w_ref, b_ref, router_ref, indices_ref):
    w = w_ref[...]
    dims = (((1,), (1,)), ((), ()))
    scores = (
        jax.lax.dot_general(w[:, :DH], xa_ref[...], dims,
                            preferred_element_type=jnp.float32)
        + jax.lax.dot_general(w[:, DH:], xb_ref[...], dims,
                              preferred_element_type=jnp.float32)
        + b_ref[...].T
    )

    eidx = jax.lax.broadcasted_iota(jnp.int32, scores.shape, 0)
    active = jnp.ones(scores.shape, dtype=jnp.bool_)
    neg_inf = jnp.float32(-jnp.inf)
    rowmax = None
    for it in range(K):
        masked = jnp.where(active, scores, neg_inf)
        m = jnp.max(masked, axis=0, keepdims=True)
        if it == 0:
            rowmax = m
        is_m = active & (scores == m)
        cand = jnp.where(is_m, eidx, E)
        j = jnp.min(cand, axis=0, keepdims=True)
        active = active & (eidx != j)
    keep = jnp.logical_not(active)

    expv = jnp.where(keep, jnp.exp(scores - rowmax), 0.0)
    router_ref[...] = expv / jnp.sum(expv, axis=0, keepdims=True)
    indices_ref[...] = keep.astype(jnp.float32)


def kernel(inputs, W, b):
    b2 = b.reshape(1, E)
    grid = (T // BT,)
    router_t, indices_t = pl.pallas_call(
        _router_block,
        grid=grid,
        in_specs=[
            pl.BlockSpec((BT, DH), lambda i: (i, 0)),
            pl.BlockSpec((BT, DH), lambda i: (i, 1)),
            pl.BlockSpec((E, D), lambda i: (0, 0)),
            pl.BlockSpec((1, E), lambda i: (0, 0)),
        ],
        out_specs=[
            pl.BlockSpec((E, BT), lambda i: (0, i)),
            pl.BlockSpec((E, BT), lambda i: (0, i)),
        ],
        out_shape=[
            jax.ShapeDtypeStruct((E, T), jnp.float32),
            jax.ShapeDtypeStruct((E, T), jnp.float32),
        ],
        compiler_params=pltpu.CompilerParams(
            dimension_semantics=("parallel",),
        ),
    )(inputs, inputs, W, b2)
    return (router_t.T, indices_t.T)


# speedup vs baseline: 1.2885x; 1.0005x over previous
import jax
import jax.numpy as jnp
from jax.experimental import pallas as pl
from jax.experimental.pallas import tpu as pltpu

T = 8192
D = 2048
E = 64
K = 8
BT = 1024
DH = D // 2


def _router_block(xa_ref, xb_ref, w_ref, b_ref, router_ref, indices_ref):
    w = w_ref[...]
    dims = (((1,), (1,)), ((), ()))
    scores = (
        jax.lax.dot_general(w[:, :DH], xa_ref[...], dims,
                            preferred_element_type=jnp.float32)
        + jax.lax.dot_general(w[:, DH:], xb_ref[...], dims,
                              preferred_element_type=jnp.float32)
        + b_ref[...].T
    )

    eidx = jax.lax.broadcasted_iota(jnp.int32, scores.shape, 0)
    active = jnp.ones(scores.shape, dtype=jnp.bool_)
    neg_inf = jnp.float32(-jnp.inf)
    rowmax = None
    for it in range(K):
        masked = jnp.where(active, scores, neg_inf)
        m = jnp.max(masked, axis=0, keepdims=True)
        if it == 0:
            rowmax = m
        is_m = active & (scores == m)
        cand = jnp.where(is_m, eidx, E)
        j = jnp.min(cand, axis=0, keepdims=True)
        active = active & (eidx != j)
    keep = jnp.logical_not(active)

    expv = jnp.where(keep, jnp.exp(scores - rowmax), 0.0)
    router_ref[...] = expv / jnp.sum(expv, axis=0, keepdims=True)
    indices_ref[...] = keep.astype(jnp.float32)


def kernel(inputs, W, b):
    b2 = b.reshape(1, E)
    grid = (T // BT,)
    router_t, indices_t = pl.pallas_call(
        _router_block,
        grid=grid,
        in_specs=[
            pl.BlockSpec((BT, DH), lambda i: (i, 0)),
            pl.BlockSpec((BT, DH), lambda i: (i, 1)),
            pl.BlockSpec((E, D), lambda i: (0, 0)),
            pl.BlockSpec((1, E), lambda i: (0, 0)),
        ],
        out_specs=[
            pl.BlockSpec((E, BT), lambda i: (0, i)),
            pl.BlockSpec((E, BT), lambda i: (0, i)),
        ],
        out_shape=[
            jax.ShapeDtypeStruct((E, T), jnp.float32),
            jax.ShapeDtypeStruct((E, T), jnp.float32),
        ],
        compiler_params=pltpu.CompilerParams(
            dimension_semantics=("parallel",),
        ),
    )(inputs, inputs, W, b2)
    return (router_t.T, indices_t.T)
